# baseline (speedup 1.0000x reference)
import functools

import jax
import jax.numpy as jnp
from jax.experimental import pallas as pl
from jax.experimental.pallas import tpu as pltpu

_K = 512


def _leaky(v):
    return jnp.where(v >= 0, v, 0.2 * v)


def _layer_kernel(src_ref, dst_ref, feat_ref, w_ref, msrc_ref, mdst_ref,
                  expand_ref, num_ref, den_ref, hb_scr, adst_scr, bound_scr):
    pid = pl.program_id(0)

    @pl.when(pid == 0)
    def _node_stage():
        h = jnp.dot(feat_ref[...], w_ref[...], preferred_element_type=jnp.float32)
        hb_scr[...] = h.astype(jnp.bfloat16)
        asrc = jnp.dot(h, msrc_ref[...], preferred_element_type=jnp.float32)
        adst = jnp.dot(h, mdst_ref[...], preferred_element_type=jnp.float32)
        adst_scr[...] = adst.astype(jnp.bfloat16)
        gmax = jnp.max(asrc)
        bound_scr[...] = _leaky(gmax + adst).astype(jnp.bfloat16)
        num_ref[...] = jnp.zeros_like(num_ref)
        den_ref[...] = jnp.zeros_like(den_ref)

    @pl.when(pid > 0)
    def _edge_stage():
        n_pad = feat_ref.shape[0]
        s_col = src_ref[0]
        d_col = dst_ref[0]
        iota = jax.lax.broadcasted_iota(jnp.int32, (_K, n_pad), 1)
        oh_s = (iota == s_col).astype(jnp.bfloat16)
        oh_d = (iota == d_col).astype(jnp.bfloat16)
        hs = jnp.dot(oh_s, hb_scr[...], preferred_element_type=jnp.float32)
        asrc_e = jnp.dot(hs, msrc_ref[...], preferred_element_type=jnp.float32)
        adst_e = jnp.dot(oh_d, adst_scr[...], preferred_element_type=jnp.float32)
        bound_e = jnp.dot(oh_d, bound_scr[...], preferred_element_type=jnp.float32)
        alpha = jnp.exp(_leaky(asrc_e + adst_e) - bound_e)
        msgs = hs * jnp.dot(alpha, expand_ref[...],
                            preferred_element_type=jnp.float32)
        dn = (((0,), (0,)), ((), ()))
        num_ref[...] += jax.lax.dot_general(oh_d, msgs.astype(jnp.bfloat16), dn,
                                            preferred_element_type=jnp.float32)
        den_ref[...] += jax.lax.dot_general(oh_d, alpha.astype(jnp.bfloat16), dn,
                                            preferred_element_type=jnp.float32)


def _run_layer(feat, w, att_src, att_dst, src_c, dst_c):
    n_pad, d_in = feat.shape
    heads, ch = att_src.shape
    hc = heads * ch
    n_chunks = src_c.shape[0]
    msrc = (att_src[:, :, None] * jnp.eye(heads, dtype=jnp.float32)[:, None, :]
            ).reshape(hc, heads)
    mdst = (att_dst[:, :, None] * jnp.eye(heads, dtype=jnp.float32)[:, None, :]
            ).reshape(hc, heads)
    expand = jnp.kron(jnp.eye(heads, dtype=jnp.float32),
                      jnp.ones((1, ch), dtype=jnp.float32))

    full = lambda shape: pl.BlockSpec(shape, lambda i: (0,) * len(shape))
    edge_spec = pl.BlockSpec((1, _K, 1), lambda i: (jnp.maximum(i - 1, 0), 0, 0))
    num, den = pl.pallas_call(
        _layer_kernel,
        grid=(n_chunks + 1,),
        in_specs=[
            edge_spec,
            edge_spec,
            full((n_pad, d_in)),
            full((d_in, hc)),
            full((hc, heads)),
            full((hc, heads)),
            full((heads, hc)),
        ],
        out_specs=[full((n_pad, hc)), full((n_pad, heads))],
        out_shape=[
            jax.ShapeDtypeStruct((n_pad, hc), jnp.float32),
            jax.ShapeDtypeStruct((n_pad, heads), jnp.float32),
        ],
        scratch_shapes=[
            pltpu.VMEM((n_pad, hc), jnp.bfloat16),
            pltpu.VMEM((n_pad, heads), jnp.bfloat16),
            pltpu.VMEM((n_pad, heads), jnp.bfloat16),
        ],
    )(src_c, dst_c, feat, w, msrc, mdst, expand)
    return num, den, expand


def _fin1_kernel(num_ref, den_ref, exp_ref, b_ref, out_ref):
    den = jnp.dot(den_ref[...], exp_ref[...], preferred_element_type=jnp.float32)
    z = num_ref[...] / (den + 1e-16) + b_ref[...]
    out_ref[...] = jnp.where(z > 0, z, jnp.exp(jnp.minimum(z, 0.0)) - 1.0)


def _fin2_kernel(num_ref, den_ref, exp_ref, b_ref, out_ref):
    den = jnp.dot(den_ref[...], exp_ref[...], preferred_element_type=jnp.float32)
    z = num_ref[...] / (den + 1e-16) + b_ref[...]
    zmax = jnp.max(z, axis=-1, keepdims=True)
    lse = jnp.log(jnp.sum(jnp.exp(z - zmax), axis=-1, keepdims=True)) + zmax
    out_ref[...] = z - lse


def _finalize(fin, num, den, expand, b):
    n_pad, hc = num.shape
    heads = den.shape[1]
    full = lambda shape: pl.BlockSpec(shape, lambda: (0,) * len(shape))
    return pl.pallas_call(
        fin,
        in_specs=[full((n_pad, hc)), full((n_pad, heads)), full((heads, hc)),
                  full((1, hc))],
        out_specs=full((n_pad, hc)),
        out_shape=jax.ShapeDtypeStruct((n_pad, hc), jnp.float32),
    )(num, den, expand, b.reshape(1, hc))


@jax.jit
def kernel(x, edge_index, W1, att_src1, att_dst1, b1, W2, att_src2, att_dst2, b2):
    n, _ = x.shape
    loop = jnp.arange(n, dtype=jnp.int32)
    src = jnp.concatenate([edge_index[0].astype(jnp.int32), loop])
    dst = jnp.concatenate([edge_index[1].astype(jnp.int32), loop])
    e_tot = src.shape[0]
    n_chunks = -(-e_tot // _K)
    pad = n_chunks * _K - e_tot
    src = jnp.pad(src, (0, pad)).reshape(n_chunks, _K, 1)
    dst = jnp.pad(dst, (0, pad), constant_values=n).reshape(n_chunks, _K, 1)

    n_pad = -(-n // 128) * 128
    feat = jnp.pad(x.astype(jnp.float32), ((0, n_pad - n), (0, 0)))

    num1, den1, exp1 = _run_layer(feat, W1, att_src1, att_dst1, src, dst)
    out1 = _finalize(_fin1_kernel, num1, den1, exp1, b1)
    num2, den2, exp2 = _run_layer(out1, W2, att_src2, att_dst2, src, dst)
    out2 = _finalize(_fin2_kernel, num2, den2, exp2, b2)
    return out2[:n]


# speedup vs baseline: 1.2412x; 1.2412x over previous
import functools

import jax
import jax.numpy as jnp
from jax.experimental import pallas as pl
from jax.experimental.pallas import tpu as pltpu

_K = 256


def _leaky(v):
    return jnp.where(v >= 0, v, 0.2 * v)


def _layer_kernel(src_ref, dst_ref, feat_ref, w_ref, msrc_ref, mdst_ref,
                  expand_ref, num_ref, den_ref, hb_scr, adst_scr, bound_scr):
    pid = pl.program_id(0)

    @pl.when(pid == 0)
    def _node_stage():
        h = jnp.dot(feat_ref[...], w_ref[...], preferred_element_type=jnp.float32)
        hb_scr[...] = h.astype(jnp.bfloat16)
        asrc = jnp.dot(h, msrc_ref[...], preferred_element_type=jnp.float32)
        adst = jnp.dot(h, mdst_ref[...], preferred_element_type=jnp.float32)
        adst_scr[...] = adst.astype(jnp.bfloat16)
        gmax = jnp.max(asrc)
        bound_scr[...] = _leaky(gmax + adst).astype(jnp.bfloat16)
        num_ref[...] = jnp.zeros_like(num_ref)
        den_ref[...] = jnp.zeros_like(den_ref)

    @pl.when(pid > 0)
    def _edge_stage():
        n_pad = feat_ref.shape[0]
        s_col = src_ref[0]
        d_col = dst_ref[0]
        iota = jax.lax.broadcasted_iota(jnp.int32, (_K, n_pad), 1)
        oh_s = (iota == s_col).astype(jnp.bfloat16)
        oh_d = (iota == d_col).astype(jnp.bfloat16)
        hs = jnp.dot(oh_s, hb_scr[...], preferred_element_type=jnp.float32)
        asrc_e = jnp.dot(hs, msrc_ref[...], preferred_element_type=jnp.float32)
        adst_e = jnp.dot(oh_d, adst_scr[...], preferred_element_type=jnp.float32)
        bound_e = jnp.dot(oh_d, bound_scr[...], preferred_element_type=jnp.float32)
        alpha = jnp.exp(_leaky(asrc_e + adst_e) - bound_e)
        msgs = hs * jnp.dot(alpha, expand_ref[...],
                            preferred_element_type=jnp.float32)
        dn = (((0,), (0,)), ((), ()))
        num_ref[...] += jax.lax.dot_general(oh_d, msgs.astype(jnp.bfloat16), dn,
                                            preferred_element_type=jnp.float32)
        den_ref[...] += jax.lax.dot_general(oh_d, alpha.astype(jnp.bfloat16), dn,
                                            preferred_element_type=jnp.float32)


def _run_layer(feat, w, att_src, att_dst, src_c, dst_c):
    n_pad, d_in = feat.shape
    heads, ch = att_src.shape
    hc = heads * ch
    n_chunks = src_c.shape[0]
    msrc = (att_src[:, :, None] * jnp.eye(heads, dtype=jnp.float32)[:, None, :]
            ).reshape(hc, heads)
    mdst = (att_dst[:, :, None] * jnp.eye(heads, dtype=jnp.float32)[:, None, :]
            ).reshape(hc, heads)
    expand = jnp.kron(jnp.eye(heads, dtype=jnp.float32),
                      jnp.ones((1, ch), dtype=jnp.float32))

    full = lambda shape: pl.BlockSpec(shape, lambda i: (0,) * len(shape))
    edge_spec = pl.BlockSpec((1, _K, 1), lambda i: (jnp.maximum(i - 1, 0), 0, 0))
    num, den = pl.pallas_call(
        _layer_kernel,
        grid=(n_chunks + 1,),
        in_specs=[
            edge_spec,
            edge_spec,
            full((n_pad, d_in)),
            full((d_in, hc)),
            full((hc, heads)),
            full((hc, heads)),
            full((heads, hc)),
        ],
        out_specs=[full((n_pad, hc)), full((n_pad, heads))],
        out_shape=[
            jax.ShapeDtypeStruct((n_pad, hc), jnp.float32),
            jax.ShapeDtypeStruct((n_pad, heads), jnp.float32),
        ],
        scratch_shapes=[
            pltpu.VMEM((n_pad, hc), jnp.bfloat16),
            pltpu.VMEM((n_pad, heads), jnp.bfloat16),
            pltpu.VMEM((n_pad, heads), jnp.bfloat16),
        ],
    )(src_c, dst_c, feat, w, msrc, mdst, expand)
    return num, den, expand


def _fin1_kernel(num_ref, den_ref, exp_ref, b_ref, out_ref):
    den = jnp.dot(den_ref[...], exp_ref[...], preferred_element_type=jnp.float32)
    z = num_ref[...] / (den + 1e-16) + b_ref[...]
    out_ref[...] = jnp.where(z > 0, z, jnp.exp(jnp.minimum(z, 0.0)) - 1.0)


def _fin2_kernel(num_ref, den_ref, exp_ref, b_ref, out_ref):
    den = jnp.dot(den_ref[...], exp_ref[...], preferred_element_type=jnp.float32)
    z = num_ref[...] / (den + 1e-16) + b_ref[...]
    zmax = jnp.max(z, axis=-1, keepdims=True)
    lse = jnp.log(jnp.sum(jnp.exp(z - zmax), axis=-1, keepdims=True)) + zmax
    out_ref[...] = z - lse


def _finalize(fin, num, den, expand, b):
    n_pad, hc = num.shape
    heads = den.shape[1]
    full = lambda shape: pl.BlockSpec(shape, lambda: (0,) * len(shape))
    return pl.pallas_call(
        fin,
        in_specs=[full((n_pad, hc)), full((n_pad, heads)), full((heads, hc)),
                  full((1, hc))],
        out_specs=full((n_pad, hc)),
        out_shape=jax.ShapeDtypeStruct((n_pad, hc), jnp.float32),
    )(num, den, expand, b.reshape(1, hc))


@jax.jit
def kernel(x, edge_index, W1, att_src1, att_dst1, b1, W2, att_src2, att_dst2, b2):
    n, _ = x.shape
    loop = jnp.arange(n, dtype=jnp.int32)
    src = jnp.concatenate([edge_index[0].astype(jnp.int32), loop])
    dst = jnp.concatenate([edge_index[1].astype(jnp.int32), loop])
    e_tot = src.shape[0]
    n_chunks = -(-e_tot // _K)
    pad = n_chunks * _K - e_tot
    src = jnp.pad(src, (0, pad)).reshape(n_chunks, _K, 1)
    dst = jnp.pad(dst, (0, pad), constant_values=n).reshape(n_chunks, _K, 1)

    n_pad = -(-n // 128) * 128
    feat = jnp.pad(x.astype(jnp.float32), ((0, n_pad - n), (0, 0)))

    num1, den1, exp1 = _run_layer(feat, W1, att_src1, att_dst1, src, dst)
    out1 = _finalize(_fin1_kernel, num1, den1, exp1, b1)
    num2, den2, exp2 = _run_layer(out1, W2, att_src2, att_dst2, src, dst)
    out2 = _finalize(_fin2_kernel, num2, den2, exp2, b2)
    return out2[:n]
